# baseline (speedup 1.0000x reference)
import jax
import jax.numpy as jnp
from jax import lax
from jax.experimental import pallas as pl
from jax.experimental.pallas import tpu as pltpu
from jax.experimental.pallas import tpu_sc as plsc

_NSEG = 1024
_N = 100000
_NW = 32
_SPW = _NSEG // _NW
_CR = 448


def _sc_body(h0, h1, h2, starts_hbm, out_hbm, starts_v, buf, res, sem0, sem1):
    nc = 2
    wid = lax.axis_index("s") * nc + lax.axis_index("c")
    base = wid * _SPW

    pltpu.sync_copy(starts_hbm.at[pl.ds(base, 64)], starts_v)

    def sval(k):
        return starts_v[pl.ds(k, 16)][0]

    lo = sval(0)
    hi = sval(_SPW)
    lo_a = (lo // 8) * 8

    neg = jnp.full((16,), -jnp.inf, jnp.float32)

    def init_body(s, _):
        for j in range(24):
            res[s, pl.ds(16 * j, 16)] = neg
        return 0

    lax.fori_loop(0, _SPW, init_body, 0)

    nch = (hi - lo_a + _CR - 1) // _CR
    npairs = (nch + 1) // 2

    def chunk_start(k):
        return jnp.minimum(lo_a + k * _CR, _N - _CR)

    for arr, h in enumerate((h0, h1, h2)):
        coff = 128 * arr

        def issue(k, parity, sem, _h=h):
            pltpu.async_copy(
                _h.at[pl.ds(chunk_start(k), _CR), :],
                buf.at[pl.ds(parity * _CR, _CR), :],
                sem,
            )

        def wait_chunk(parity, sem, _h=h):
            pltpu.make_async_copy(
                _h.at[pl.ds(0, _CR), :],
                buf.at[pl.ds(parity * _CR, _CR), :],
                sem,
            ).wait()

        def process(k, parity, _coff=coff):
            cb = chunk_start(k)
            c1 = cb + _CR
            soff = parity * _CR - 0

            def seg_body(s, _):
                a = jnp.maximum(sval(s), cb)
                b = jnp.minimum(sval(s + 1), c1)
                n = b - a

                def maxrow(rg, row):
                    return tuple(
                        jnp.maximum(rg[j], buf[row, pl.ds(16 * j, 16)])
                        for j in range(8)
                    )

                def work(_i, _c):
                    regs = tuple(
                        res[s, pl.ds(_coff + 16 * j, 16)] for j in range(8))

                    def u8body(i, rg):
                        row0 = soff + (a - cb) + 8 * i
                        for k in range(8):
                            rg = maxrow(rg, row0 + k)
                        return rg

                    regs = lax.fori_loop(0, n // 8, u8body, regs)
                    for k in range(8):
                        rr = jnp.maximum(b - 8 + k, a)
                        regs = maxrow(regs, soff + (rr - cb))
                    for j in range(8):
                        res[s, pl.ds(_coff + 16 * j, 16)] = regs[j]
                    return 0

                lax.fori_loop(0, (n > 0).astype(jnp.int32), work, 0)
                return 0

            lax.fori_loop(0, _SPW, seg_body, 0)

        issue(jnp.int32(0), 0, sem0)
        issue(jnp.int32(1), 1, sem1)

        def pair_body(p, _):
            k0 = 2 * p
            wait_chunk(0, sem0)
            process(k0, 0)
            issue(k0 + 2, 0, sem0)
            wait_chunk(1, sem1)
            process(k0 + 1, 1)
            issue(k0 + 3, 1, sem1)
            return 0

        lax.fori_loop(0, npairs, pair_body, 0)

        wait_chunk(0, sem0)
        wait_chunk(1, sem1)

    pltpu.sync_copy(res, out_hbm.at[pl.ds(base, _SPW), :])


def kernel(h0, h1, h2, index):
    idx32 = index.astype(jnp.int32)
    targets = jnp.arange(_NSEG + 1, dtype=jnp.int32)
    blk = idx32.reshape(1000, 100)
    mins = blk[:, 0]
    nb = jnp.sum((mins[None, :] < targets[:, None]), axis=1, dtype=jnp.int32)
    b = jnp.maximum(nb - 1, 0)
    rows = blk[b]
    within = jnp.sum(rows < targets[:, None], axis=1, dtype=jnp.int32)
    starts = (100 * b + within).astype(jnp.int32)
    starts = jnp.concatenate(
        [starts, jnp.full((39,), jnp.int32(_N))])

    mesh = plsc.VectorSubcoreMesh(
        core_axis_name="c", subcore_axis_name="s", num_cores=2, num_subcores=16)
    f = pl.kernel(
        _sc_body,
        out_type=jax.ShapeDtypeStruct((_NSEG, 384), jnp.float32),
        mesh=mesh,
        scratch_types=[
            pltpu.VMEM((64,), jnp.int32),
            pltpu.VMEM((2 * _CR, 128), jnp.float32),
            pltpu.VMEM((_SPW, 384), jnp.float32),
            pltpu.SemaphoreType.DMA,
            pltpu.SemaphoreType.DMA,
        ],
    )
    return f(h0, h1, h2, starts)


# speedup vs baseline: 1.0809x; 1.0809x over previous
import jax
import jax.numpy as jnp
from jax import lax
from jax.experimental import pallas as pl
from jax.experimental.pallas import tpu as pltpu
from jax.experimental.pallas import tpu_sc as plsc

_NSEG = 1024
_N = 100000
_NW = 32
_SPW = _NSEG // _NW
_CR = 448


def _sc_body(h0, h1, h2, starts_hbm, out_hbm, starts_v, buf, res, sem0, sem1):
    nc = 2
    wid = lax.axis_index("s") * nc + lax.axis_index("c")
    base = wid * _SPW

    pltpu.sync_copy(starts_hbm.at[pl.ds(base, 64)], starts_v)

    def sval(k):
        return starts_v[pl.ds(k, 16)][0]

    lo = sval(0)
    hi = sval(_SPW)
    lo_a = (lo // 8) * 8

    neg = jnp.full((16,), -jnp.inf, jnp.float32)

    def init_body(s, _):
        for j in range(24):
            res[s, pl.ds(16 * j, 16)] = neg
        return 0

    lax.fori_loop(0, _SPW, init_body, 0)

    nch = (hi - lo_a + _CR - 1) // _CR
    npairs = (nch + 1) // 2

    def chunk_start(k):
        return jnp.minimum(lo_a + k * _CR, _N - _CR)

    for arr, h in enumerate((h0, h1, h2)):
        coff = 128 * arr

        def issue(k, parity, sem, _h=h):
            def full(_i, _c):
                pltpu.async_copy(
                    _h.at[pl.ds(chunk_start(k), _CR), :],
                    buf.at[pl.ds(parity * _CR, _CR), :],
                    sem,
                )
                return 0

            def small(_i, _c):
                pltpu.async_copy(
                    _h.at[pl.ds(0, 8), :],
                    buf.at[pl.ds(parity * _CR, 8), :],
                    sem,
                )
                return 0

            real = (k < nch).astype(jnp.int32)
            lax.fori_loop(0, real, full, 0)
            lax.fori_loop(0, 1 - real, small, 0)

        def wait_chunk(k, parity, sem, _h=h):
            def full(_i, _c):
                pltpu.make_async_copy(
                    _h.at[pl.ds(0, _CR), :],
                    buf.at[pl.ds(parity * _CR, _CR), :],
                    sem,
                ).wait()
                return 0

            def small(_i, _c):
                pltpu.make_async_copy(
                    _h.at[pl.ds(0, 8), :],
                    buf.at[pl.ds(parity * _CR, 8), :],
                    sem,
                ).wait()
                return 0

            real = (k < nch).astype(jnp.int32)
            lax.fori_loop(0, real, full, 0)
            lax.fori_loop(0, 1 - real, small, 0)

        def process(k, parity, _coff=coff):
            cb = chunk_start(k)
            c1 = cb + _CR * (k < nch).astype(jnp.int32)
            soff = parity * _CR - 0

            def seg_body(s, _):
                a = jnp.maximum(sval(s), cb)
                b = jnp.minimum(sval(s + 1), c1)
                n = b - a

                def maxrow(rg, row):
                    return tuple(
                        jnp.maximum(rg[j], buf[row, pl.ds(16 * j, 16)])
                        for j in range(8)
                    )

                def work(_i, _c):
                    regs = tuple(
                        res[s, pl.ds(_coff + 16 * j, 16)] for j in range(8))

                    def u8body(i, rg):
                        row0 = soff + (a - cb) + 8 * i
                        for k in range(8):
                            rg = maxrow(rg, row0 + k)
                        return rg

                    regs = lax.fori_loop(0, n // 8, u8body, regs)
                    for k in range(8):
                        rr = jnp.maximum(b - 8 + k, a)
                        regs = maxrow(regs, soff + (rr - cb))
                    for j in range(8):
                        res[s, pl.ds(_coff + 16 * j, 16)] = regs[j]
                    return 0

                lax.fori_loop(0, (n > 0).astype(jnp.int32), work, 0)
                return 0

            lax.fori_loop(0, _SPW, seg_body, 0)

        issue(jnp.int32(0), 0, sem0)
        issue(jnp.int32(1), 1, sem1)

        def pair_body(p, _):
            k0 = 2 * p
            wait_chunk(k0, 0, sem0)
            process(k0, 0)
            issue(k0 + 2, 0, sem0)
            wait_chunk(k0 + 1, 1, sem1)
            process(k0 + 1, 1)
            issue(k0 + 3, 1, sem1)
            return 0

        lax.fori_loop(0, npairs, pair_body, 0)

        wait_chunk(2 * npairs, 0, sem0)
        wait_chunk(2 * npairs + 1, 1, sem1)

    pltpu.sync_copy(res, out_hbm.at[pl.ds(base, _SPW), :])


def kernel(h0, h1, h2, index):
    idx32 = index.astype(jnp.int32)
    targets = jnp.arange(_NSEG + 1, dtype=jnp.int32)
    blk = idx32.reshape(1000, 100)
    mins = blk[:, 0]
    nb = jnp.sum((mins[None, :] < targets[:, None]), axis=1, dtype=jnp.int32)
    b = jnp.maximum(nb - 1, 0)
    rows = blk[b]
    within = jnp.sum(rows < targets[:, None], axis=1, dtype=jnp.int32)
    starts = (100 * b + within).astype(jnp.int32)
    starts = jnp.concatenate(
        [starts, jnp.full((39,), jnp.int32(_N))])

    mesh = plsc.VectorSubcoreMesh(
        core_axis_name="c", subcore_axis_name="s", num_cores=2, num_subcores=16)
    f = pl.kernel(
        _sc_body,
        out_type=jax.ShapeDtypeStruct((_NSEG, 384), jnp.float32),
        mesh=mesh,
        scratch_types=[
            pltpu.VMEM((64,), jnp.int32),
            pltpu.VMEM((2 * _CR, 128), jnp.float32),
            pltpu.VMEM((_SPW, 384), jnp.float32),
            pltpu.SemaphoreType.DMA,
            pltpu.SemaphoreType.DMA,
        ],
    )
    return f(h0, h1, h2, starts)
